# baseline (speedup 1.0000x reference)
import functools

import jax
import jax.numpy as jnp
from jax.experimental import pallas as pl
from jax.experimental.pallas import tpu as pltpu

_DIM = 64
_MAX_LENGTH = 160
_TBL = 2 * _MAX_LENGTH


def _body(tbl_ref, out_ref, ctv, *, seq):
    pid = pl.program_id(0)

    @pl.when(pid == 0)
    def _build():
        crows = 2 * seq
        s = jax.lax.broadcasted_iota(jnp.int32, (_TBL, crows), 1)
        k = jax.lax.broadcasted_iota(jnp.int32, (_TBL, crows), 0)
        idx = jnp.clip(seq + _MAX_LENGTH - 1 - s, 0, _TBL - 1)
        g = (k == idx).astype(jnp.float32)
        ctv[...] = jax.lax.dot_general(
            tbl_ref[...], g, (((0,), (0,)), ((), ())),
            precision=jax.lax.Precision.HIGHEST,
            preferred_element_type=jnp.float32)

    m = seq - 1 - pid
    a = m // 128
    p = m - a * 128
    w = ctv[:, pl.ds(pl.multiple_of(a * 128, 128), seq + 128)]
    rolled = pltpu.roll(w, (seq + 128 - p) % (seq + 128), 1)
    for jt in range(seq // 128):
        sl = rolled[:, 128 * jt:128 * (jt + 1)]
        out_ref[0, :, jt] = sl.reshape(8, 8, 128)


def kernel(hidden_states, pe_k_weight):
    seq = hidden_states.shape[1]
    out = pl.pallas_call(
        functools.partial(_body, seq=seq),
        grid=(seq,),
        in_specs=[pl.BlockSpec((_TBL, _DIM), lambda i: (0, 0))],
        out_specs=pl.BlockSpec(
            (1, 8, seq // 128, 8, 128), lambda i: (i, 0, 0, 0, 0)),
        out_shape=jax.ShapeDtypeStruct(
            (seq, 8, seq // 128, 8, 128), jnp.float32),
        scratch_shapes=[
            pltpu.VMEM((_DIM, 2 * seq), jnp.float32),
        ],
        compiler_params=pltpu.CompilerParams(
            dimension_semantics=("arbitrary",)),
    )(pe_k_weight)
    return out.transpose(0, 2, 4, 1, 3).reshape(seq, seq, _DIM)
